# baseline (speedup 1.0000x reference)
import functools

import jax
import jax.numpy as jnp
from jax.experimental import pallas as pl
from jax.experimental.pallas import tpu as pltpu

EPS_DIAG = 1e-06
T_TILE = 4096


def _cln_kernel(xr_ref, xi_ref, wb_ref, yr_ref, yi_ref):
    xr = xr_ref[0]
    xi = xi_ref[0]
    c_dim = xr.shape[0]
    inv_c = 1.0 / c_dim

    mu_r = jnp.sum(xr, axis=0, keepdims=True) * inv_c
    mu_i = jnp.sum(xi, axis=0, keepdims=True) * inv_c
    xc_r = xr - mu_r
    xc_i = xi - mu_i

    a = jnp.sum(xc_r * xc_r, axis=0, keepdims=True) * inv_c + EPS_DIAG
    b = jnp.sum(xc_r * xc_i, axis=0, keepdims=True) * inv_c
    c = jnp.sum(xc_i * xc_i, axis=0, keepdims=True) * inv_c + EPS_DIAG

    det = a * c - b * b
    r1 = jax.lax.rsqrt(det)
    s = det * r1
    r2 = jax.lax.rsqrt(a + c + 2.0 * s)
    f = r1 * r2
    w00 = (c + s) * f
    w01 = -b * f
    w11 = (a + s) * f

    z_r = xc_r * w00 + xc_i * w01
    z_i = xc_r * w01 + xc_i * w11

    g00 = wb_ref[:, 0:1]
    g01 = wb_ref[:, 1:2]
    g10 = wb_ref[:, 2:3]
    g11 = wb_ref[:, 3:4]
    b0 = wb_ref[:, 4:5]
    b1 = wb_ref[:, 5:6]
    yr_ref[0] = z_r * g00 + z_i * g10 + b0
    yi_ref[0] = z_r * g01 + z_i * g11 + b1


@jax.jit
def kernel(x_real, x_imag, weight, bias):
    B, C, T = x_real.shape
    w4 = jnp.transpose(weight, (2, 0, 1)).reshape(C, 4)
    b2 = jnp.transpose(bias, (1, 0))
    wb = jnp.concatenate([w4, b2], axis=1)

    grid = (B, T // T_TILE)
    x_spec = pl.BlockSpec((1, C, T_TILE), lambda b, t: (b, 0, t))
    wb_spec = pl.BlockSpec((C, 6), lambda b, t: (0, 0))

    yr, yi = pl.pallas_call(
        _cln_kernel,
        grid=grid,
        in_specs=[x_spec, x_spec, wb_spec],
        out_specs=[x_spec, x_spec],
        out_shape=[
            jax.ShapeDtypeStruct((B, C, T), x_real.dtype),
            jax.ShapeDtypeStruct((B, C, T), x_real.dtype),
        ],
        compiler_params=pltpu.CompilerParams(
            dimension_semantics=("parallel", "parallel")
        ),
    )(x_real, x_imag, wb)
    return yr, yi


# speedup vs baseline: 1.0082x; 1.0082x over previous
import functools

import jax
import jax.numpy as jnp
from jax.experimental import pallas as pl
from jax.experimental.pallas import tpu as pltpu

EPS_DIAG = 1e-06
T_TILE = 1024


def _cln_kernel(xr_ref, xi_ref, wb_ref, yr_ref, yi_ref):
    xr = xr_ref[0]
    xi = xi_ref[0]
    c_dim = xr.shape[0]
    inv_c = 1.0 / c_dim

    mu_r = jnp.sum(xr, axis=0, keepdims=True) * inv_c
    mu_i = jnp.sum(xi, axis=0, keepdims=True) * inv_c
    xc_r = xr - mu_r
    xc_i = xi - mu_i

    a = jnp.sum(xc_r * xc_r, axis=0, keepdims=True) * inv_c + EPS_DIAG
    b = jnp.sum(xc_r * xc_i, axis=0, keepdims=True) * inv_c
    c = jnp.sum(xc_i * xc_i, axis=0, keepdims=True) * inv_c + EPS_DIAG

    det = a * c - b * b
    r1 = jax.lax.rsqrt(det)
    s = det * r1
    r2 = jax.lax.rsqrt(a + c + 2.0 * s)
    f = r1 * r2
    w00 = (c + s) * f
    w01 = -b * f
    w11 = (a + s) * f

    z_r = xc_r * w00 + xc_i * w01
    z_i = xc_r * w01 + xc_i * w11

    g00 = wb_ref[:, 0:1]
    g01 = wb_ref[:, 1:2]
    g10 = wb_ref[:, 2:3]
    g11 = wb_ref[:, 3:4]
    b0 = wb_ref[:, 4:5]
    b1 = wb_ref[:, 5:6]
    yr_ref[0] = z_r * g00 + z_i * g10 + b0
    yi_ref[0] = z_r * g01 + z_i * g11 + b1


@jax.jit
def kernel(x_real, x_imag, weight, bias):
    B, C, T = x_real.shape
    w4 = jnp.transpose(weight, (2, 0, 1)).reshape(C, 4)
    b2 = jnp.transpose(bias, (1, 0))
    wb = jnp.concatenate([w4, b2], axis=1)

    grid = (B, T // T_TILE)
    x_spec = pl.BlockSpec((1, C, T_TILE), lambda b, t: (b, 0, t))
    wb_spec = pl.BlockSpec((C, 6), lambda b, t: (0, 0))

    yr, yi = pl.pallas_call(
        _cln_kernel,
        grid=grid,
        in_specs=[x_spec, x_spec, wb_spec],
        out_specs=[x_spec, x_spec],
        out_shape=[
            jax.ShapeDtypeStruct((B, C, T), x_real.dtype),
            jax.ShapeDtypeStruct((B, C, T), x_real.dtype),
        ],
        compiler_params=pltpu.CompilerParams(
            dimension_semantics=("parallel", "parallel")
        ),
    )(x_real, x_imag, wb)
    return yr, yi


# speedup vs baseline: 1.1490x; 1.1397x over previous
import functools

import jax
import jax.numpy as jnp
from jax.experimental import pallas as pl
from jax.experimental.pallas import tpu as pltpu

EPS_DIAG = 1e-06
T_TILE = 2048


def _cln_kernel(xr_ref, xi_ref, wb_ref, yr_ref, yi_ref):
    xr = xr_ref[0]
    xi = xi_ref[0]
    c_dim = xr.shape[0]
    inv_c = 1.0 / c_dim

    mu_r = jnp.sum(xr, axis=0, keepdims=True) * inv_c
    mu_i = jnp.sum(xi, axis=0, keepdims=True) * inv_c
    xc_r = xr - mu_r
    xc_i = xi - mu_i

    a = jnp.sum(xc_r * xc_r, axis=0, keepdims=True) * inv_c + EPS_DIAG
    b = jnp.sum(xc_r * xc_i, axis=0, keepdims=True) * inv_c
    c = jnp.sum(xc_i * xc_i, axis=0, keepdims=True) * inv_c + EPS_DIAG

    det = a * c - b * b
    r1 = jax.lax.rsqrt(det)
    s = det * r1
    r2 = jax.lax.rsqrt(a + c + 2.0 * s)
    f = r1 * r2
    w00 = (c + s) * f
    w01 = -b * f
    w11 = (a + s) * f

    z_r = xc_r * w00 + xc_i * w01
    z_i = xc_r * w01 + xc_i * w11

    g00 = wb_ref[:, 0:1]
    g01 = wb_ref[:, 1:2]
    g10 = wb_ref[:, 2:3]
    g11 = wb_ref[:, 3:4]
    b0 = wb_ref[:, 4:5]
    b1 = wb_ref[:, 5:6]
    yr_ref[0] = z_r * g00 + z_i * g10 + b0
    yi_ref[0] = z_r * g01 + z_i * g11 + b1


@jax.jit
def kernel(x_real, x_imag, weight, bias):
    B, C, T = x_real.shape
    w4 = jnp.transpose(weight, (2, 0, 1)).reshape(C, 4)
    b2 = jnp.transpose(bias, (1, 0))
    wb = jnp.concatenate([w4, b2], axis=1)

    grid = (B, T // T_TILE)
    x_spec = pl.BlockSpec((1, C, T_TILE), lambda b, t: (b, 0, t))
    wb_spec = pl.BlockSpec((C, 6), lambda b, t: (0, 0))

    yr, yi = pl.pallas_call(
        _cln_kernel,
        grid=grid,
        in_specs=[x_spec, x_spec, wb_spec],
        out_specs=[x_spec, x_spec],
        out_shape=[
            jax.ShapeDtypeStruct((B, C, T), x_real.dtype),
            jax.ShapeDtypeStruct((B, C, T), x_real.dtype),
        ],
        compiler_params=pltpu.CompilerParams(
            dimension_semantics=("parallel", "parallel")
        ),
    )(x_real, x_imag, wb)
    return yr, yi
